# baseline (speedup 1.0000x reference)
import functools

import jax
import jax.numpy as jnp
from jax import lax
from jax.experimental import pallas as pl
from jax.experimental.pallas import tpu as pltpu
from jax.experimental.pallas import tpu_sc as plsc

EMBED = 64
COMB = 2 * EMBED
NCORES = 2
NSUB = 16
NWORKERS = NCORES * NSUB
CHUNK = 64
TILE = 4096


def _sc_gather(idx_all, B, table, w2):
    per_w = B // NWORKERS
    n_chunks = per_w // CHUNK
    assert n_chunks % 2 == 0
    rows_out = jax.ShapeDtypeStruct((3 * B, COMB), jnp.float32)
    vec_out = jax.ShapeDtypeStruct((2 * B,), jnp.float32)
    mesh = plsc.VectorSubcoreMesh(
        core_axis_name="c", subcore_axis_name="s",
        num_cores=NCORES, num_subcores=NSUB)

    rbuf = pltpu.VMEM((CHUNK, COMB), jnp.float32)
    stage = pltpu.VMEM((per_w,), jnp.float32)

    @functools.partial(
        pl.kernel,
        out_type=[rows_out, vec_out],
        mesh=mesh,
        compiler_params=pltpu.CompilerParams(
            needs_layout_passes=False, use_tc_tiling_on_sc=True),
        scratch_types=[
            pltpu.VMEM((per_w,), jnp.int32),
            pltpu.VMEM((per_w,), jnp.int32),
            pltpu.VMEM((per_w,), jnp.int32),
            [rbuf] * 3,
            [rbuf] * 3,
            [stage] * 2,
            pltpu.SemaphoreType.DMA,
            pltpu.SemaphoreType.DMA,
            pltpu.SemaphoreType.DMA,
        ],
    )
    def k(idx_h, table_h, w2_h, o_rows, o_w,
          idx_u, idx_p, idx_n, bufa, bufb, st, sem_g, sem_o, sem_s):
        wid = lax.axis_index("s") * NCORES + lax.axis_index("c")
        base = wid * per_w
        pltpu.sync_copy(idx_h.at[pl.ds(base, per_w)], idx_u)
        pltpu.sync_copy(idx_h.at[pl.ds(B + base, per_w)], idx_p)
        pltpu.sync_copy(idx_h.at[pl.ds(2 * B + base, per_w)], idx_n)

        pltpu.async_copy(w2_h.at[idx_p.at[...]], st[0], sem_s)
        pltpu.async_copy(w2_h.at[idx_n.at[...]], st[1], sem_s)

        idxs = (idx_u, idx_p, idx_n)

        def issue_gather(g, bufs):
            s = pl.ds(g * CHUNK, CHUNK)
            for ix, b in zip(idxs, bufs):
                pltpu.async_copy(table_h.at[ix.at[s]], b, sem_g)

        def wait_gather(bufs):
            s0 = pl.ds(0, CHUNK)
            for ix, b in zip(idxs, bufs):
                pltpu.make_async_copy(table_h.at[ix.at[s0]], b, sem_g).wait()

        def issue_out(g, bufs):
            for j, b in enumerate(bufs):
                d = pl.ds(j * B + base + g * CHUNK, CHUNK)
                pltpu.async_copy(b, o_rows.at[d], sem_o)

        def wait_out(bufs):
            d = pl.ds(base, CHUNK)
            for b in bufs:
                pltpu.make_async_copy(b, o_rows.at[d], sem_o).wait()

        issue_gather(0, bufa)
        issue_gather(1, bufb)

        def pair(t, _):
            g0 = 2 * t
            g1 = 2 * t + 1
            wait_gather(bufa)
            issue_out(g0, bufa)
            wait_gather(bufb)
            issue_out(g1, bufb)
            wait_out(bufa)

            @pl.when(g0 + 2 < n_chunks)
            def _():
                issue_gather(g0 + 2, bufa)

            wait_out(bufb)

            @pl.when(g1 + 2 < n_chunks)
            def _():
                issue_gather(g1 + 2, bufb)

            return 0

        lax.fori_loop(0, n_chunks // 2, pair, 0)

        pltpu.make_async_copy(w2_h.at[idx_p.at[...]], st[0], sem_s).wait()
        pltpu.make_async_copy(w2_h.at[idx_n.at[...]], st[1], sem_s).wait()
        pltpu.sync_copy(st[0], o_w.at[pl.ds(base, per_w)])
        pltpu.sync_copy(st[1], o_w.at[pl.ds(B + base, per_w)])

    return k(idx_all, table, w2)


def _log_sigmoid(x):
    return jnp.minimum(x, 0.0) - jnp.log1p(jnp.exp(-jnp.abs(x)))


def _softplus(x):
    return jnp.maximum(x, 0.0) + jnp.log1p(jnp.exp(-jnp.abs(x)))


def _loss_body(inv_b, u_ref, p_ref, n_ref, wp_ref, wn_ref, m_ref, o_ref):
    u = u_ref[...]
    p = p_ref[...]
    n = n_ref[...]
    pi = jnp.sum(u[:, :EMBED] * p[:, :EMBED], axis=1)
    ni = jnp.sum(u[:, :EMBED] * n[:, :EMBED], axis=1)
    pp = jnp.sum(u[:, EMBED:] * p[:, EMBED:], axis=1)
    np_ = jnp.sum(u[:, EMBED:] * n[:, EMBED:], axis=1)
    m = m_ref[...]
    loss_int = -jnp.sum(m * _log_sigmoid(pi - ni))
    loss_pop = -(jnp.sum(m * _log_sigmoid(np_ - pp))
                 + jnp.sum((1.0 - m) * _log_sigmoid(pp - np_)))
    p_tide = wp_ref[...] * (pi + pp)
    n_tide = wn_ref[...] * (ni + np_)
    loss_tide = -jnp.sum(_log_sigmoid(p_tide - n_tide))
    part = (0.1 * loss_int + 0.1 * loss_pop + 0.2 * loss_tide) * inv_b

    @pl.when(pl.program_id(0) == 0)
    def _():
        o_ref[...] = jnp.zeros((1, 1), jnp.float32)

    o_ref[...] += jnp.reshape(part, (1, 1))


def _loss_tc(rows, wvec, maskf, B):
    nb = B // TILE
    row_spec = lambda off: pl.BlockSpec((TILE, COMB), lambda i: (i + off, 0))
    vec_spec = lambda off: pl.BlockSpec((TILE,), lambda i: (i + off,))
    return pl.pallas_call(
        functools.partial(_loss_body, 1.0 / B),
        grid=(nb,),
        in_specs=[row_spec(0), row_spec(nb), row_spec(2 * nb),
                  vec_spec(0), vec_spec(nb), vec_spec(0)],
        out_specs=pl.BlockSpec((1, 1), lambda i: (0, 0)),
        out_shape=jax.ShapeDtypeStruct((1, 1), jnp.float32),
    )(rows, rows, rows, wvec, wvec, maskf)


def kernel(user, item_p, item_n, mask, users_int, users_pop, items_int, items_pop, q, b):
    B = user.size
    n_items = items_int.shape[0]
    idx_all = jnp.concatenate(
        [user.reshape(-1), item_p.reshape(-1) + n_items,
         item_n.reshape(-1) + n_items])
    table = jnp.concatenate(
        [jnp.concatenate([users_int, users_pop], axis=1),
         jnp.concatenate([items_int, items_pop], axis=1)], axis=0)
    w = jnp.tanh(_softplus(q) + _softplus(b))
    w2 = jnp.concatenate([w, w])
    rows, wvec = _sc_gather(idx_all, B, table, w2)
    maskf = mask.reshape(-1).astype(jnp.float32)
    loss = _loss_tc(rows, wvec, maskf, B)
    return loss.reshape(())


# speedup vs baseline: 1.0721x; 1.0721x over previous
import functools

import jax
import jax.numpy as jnp
from jax import lax
from jax.experimental import pallas as pl
from jax.experimental.pallas import tpu as pltpu
from jax.experimental.pallas import tpu_sc as plsc

EMBED = 64
COMB = 2 * EMBED
NCORES = 2
NSUB = 16
NWORKERS = NCORES * NSUB
CHUNK = 64
TILE = 4096


def _sc_gather(idx_all, B, table, w2):
    per_w = B // NWORKERS
    n_chunks = per_w // CHUNK
    assert n_chunks % 2 == 0
    rows_out = jax.ShapeDtypeStruct((3 * B, COMB), jnp.float32)
    vec_out = jax.ShapeDtypeStruct((2 * B,), jnp.float32)
    mesh = plsc.VectorSubcoreMesh(
        core_axis_name="c", subcore_axis_name="s",
        num_cores=NCORES, num_subcores=NSUB)

    rbuf = pltpu.VMEM((CHUNK, COMB), jnp.float32)
    stage = pltpu.VMEM((per_w,), jnp.float32)

    @functools.partial(
        pl.kernel,
        out_type=[rows_out, vec_out],
        mesh=mesh,
        compiler_params=pltpu.CompilerParams(
            needs_layout_passes=False, use_tc_tiling_on_sc=True),
        scratch_types=[
            pltpu.VMEM((per_w,), jnp.int32),
            pltpu.VMEM((per_w,), jnp.int32),
            pltpu.VMEM((per_w,), jnp.int32),
            [rbuf] * 3,
            [rbuf] * 3,
            [stage] * 2,
            pltpu.SemaphoreType.DMA,
            pltpu.SemaphoreType.DMA,
            pltpu.SemaphoreType.DMA,
        ],
    )
    def k(idx_h, table_h, w2_h, o_rows, o_w,
          idx_u, idx_p, idx_n, bufa, bufb, st, sem_g, sem_o, sem_s):
        wid = lax.axis_index("s") * NCORES + lax.axis_index("c")
        base = wid * per_w
        pltpu.sync_copy(idx_h.at[pl.ds(base, per_w)], idx_u)
        pltpu.sync_copy(idx_h.at[pl.ds(B + base, per_w)], idx_p)
        pltpu.sync_copy(idx_h.at[pl.ds(2 * B + base, per_w)], idx_n)

        pltpu.async_copy(w2_h.at[idx_p.at[...]], st[0], sem_s)
        pltpu.async_copy(w2_h.at[idx_n.at[...]], st[1], sem_s)

        idxs = (idx_u, idx_p, idx_n)

        def issue_gather(g, bufs):
            s = pl.ds(g * CHUNK, CHUNK)
            for ix, b in zip(idxs, bufs):
                pltpu.async_copy(table_h.at[ix.at[s]], b, sem_g)

        def wait_gather(bufs):
            s0 = pl.ds(0, CHUNK)
            for ix, b in zip(idxs, bufs):
                pltpu.make_async_copy(table_h.at[ix.at[s0]], b, sem_g).wait()

        def issue_out(g, bufs):
            for j, b in enumerate(bufs):
                d = pl.ds(j * B + base + g * CHUNK, CHUNK)
                pltpu.async_copy(b, o_rows.at[d], sem_o)

        def wait_out(bufs):
            d = pl.ds(base, CHUNK)
            for b in bufs:
                pltpu.make_async_copy(b, o_rows.at[d], sem_o).wait()

        issue_gather(0, bufa)
        issue_gather(1, bufb)

        def pair(t, _):
            g0 = 2 * t
            g1 = 2 * t + 1
            wait_gather(bufa)
            issue_out(g0, bufa)
            wait_gather(bufb)
            issue_out(g1, bufb)
            wait_out(bufa)

            @pl.when(g0 + 2 < n_chunks)
            def _():
                issue_gather(g0 + 2, bufa)

            wait_out(bufb)

            @pl.when(g1 + 2 < n_chunks)
            def _():
                issue_gather(g1 + 2, bufb)

            return 0

        lax.fori_loop(0, n_chunks // 2, pair, 0)

        pltpu.make_async_copy(w2_h.at[idx_p.at[...]], st[0], sem_s).wait()
        pltpu.make_async_copy(w2_h.at[idx_n.at[...]], st[1], sem_s).wait()
        pltpu.sync_copy(st[0], o_w.at[pl.ds(base, per_w)])
        pltpu.sync_copy(st[1], o_w.at[pl.ds(B + base, per_w)])

    return k(idx_all, table, w2)


def _log_sigmoid(x):
    return jnp.minimum(x, 0.0) - jnp.log1p(jnp.exp(-jnp.abs(x)))


def _softplus(x):
    return jnp.maximum(x, 0.0) + jnp.log1p(jnp.exp(-jnp.abs(x)))


def _loss_body(inv_b, u_ref, p_ref, n_ref, wp_ref, wn_ref, m_ref, o_ref):
    u = u_ref[...]
    p = p_ref[...]
    n = n_ref[...]
    pi = jnp.sum(u[:, :EMBED] * p[:, :EMBED], axis=1)
    ni = jnp.sum(u[:, :EMBED] * n[:, :EMBED], axis=1)
    pp = jnp.sum(u[:, EMBED:] * p[:, EMBED:], axis=1)
    np_ = jnp.sum(u[:, EMBED:] * n[:, EMBED:], axis=1)
    m = m_ref[...]
    loss_int = -jnp.sum(m * _log_sigmoid(pi - ni))
    loss_pop = -(jnp.sum(m * _log_sigmoid(np_ - pp))
                 + jnp.sum((1.0 - m) * _log_sigmoid(pp - np_)))
    p_tide = wp_ref[...] * (pi + pp)
    n_tide = wn_ref[...] * (ni + np_)
    loss_tide = -jnp.sum(_log_sigmoid(p_tide - n_tide))
    part = (0.1 * loss_int + 0.1 * loss_pop + 0.2 * loss_tide) * inv_b

    @pl.when(pl.program_id(0) == 0)
    def _():
        o_ref[...] = jnp.zeros((1, 1), jnp.float32)

    o_ref[...] += jnp.reshape(part, (1, 1))


def _loss_tc(rows, wvec, maskf, bh, inv_b):
    nb = bh // TILE
    row_spec = lambda off: pl.BlockSpec((TILE, COMB), lambda i: (i + off, 0))
    vec_spec = lambda off: pl.BlockSpec((TILE,), lambda i: (i + off,))
    return pl.pallas_call(
        functools.partial(_loss_body, inv_b),
        grid=(nb,),
        in_specs=[row_spec(0), row_spec(nb), row_spec(2 * nb),
                  vec_spec(0), vec_spec(nb), vec_spec(0)],
        out_specs=pl.BlockSpec((1, 1), lambda i: (0, 0)),
        out_shape=jax.ShapeDtypeStruct((1, 1), jnp.float32),
    )(rows, rows, rows, wvec, wvec, maskf)


def kernel(user, item_p, item_n, mask, users_int, users_pop, items_int, items_pop, q, b):
    B = user.size
    H = B // 2
    n_items = items_int.shape[0]
    u = user.reshape(-1)
    p = item_p.reshape(-1) + n_items
    n = item_n.reshape(-1) + n_items
    table = jnp.concatenate(
        [jnp.concatenate([users_int, users_pop], axis=1),
         jnp.concatenate([items_int, items_pop], axis=1)], axis=0)
    w = jnp.tanh(_softplus(q) + _softplus(b))
    w2 = jnp.concatenate([w, w])
    maskf = mask.reshape(-1).astype(jnp.float32)
    loss = 0.0
    for s in (slice(0, H), slice(H, B)):
        idx_h = jnp.concatenate([u[s], p[s], n[s]])
        rows, wvec = _sc_gather(idx_h, H, table, w2)
        loss = loss + _loss_tc(rows, wvec, maskf[s], H, 1.0 / B)
    return loss.reshape(())
